# Initial kernel scaffold; baseline (speedup 1.0000x reference)
import jax
import jax.numpy as jnp
from jax.experimental import pallas as pl


def kernel(seq_embedding, x, edge_index, batch, embed_W, embed_b, gat_W, gat_as, gat_ad, gat_b, ln_g, ln_b, out1_W, out1_b, out2_W, out2_b, seq_W, seq_b, str_W, str_b, mha1_Wi, mha1_bi, mha1_Wo, mha1_bo, mha2_Wi, mha2_bi, mha2_Wo, mha2_bo, p1_W, p1_b, p2_W, p2_b, p3_W, p3_b):
    raise NotImplementedError("write your pallas kernel here")


# speedup vs baseline: 57.2344x; 57.2344x over previous
import functools

import jax
import jax.numpy as jnp
from jax import lax
from jax.experimental import pallas as pl
from jax.experimental.pallas import tpu as pltpu
from jax.experimental.pallas import tpu_sc as plsc

N = 10000
E = 320000
B = 64
HID = 128
HEADS = 4
DH = 32
HD = 256
SEQ = 768
LAYERS = 4

NP = 10112
EW = 144
C = 128
CREAL = 125
NCHUNK = 80
NTILES = 32
ROWS_PER_TILE = NP // 16
DUMMY_DST = N + 8

_f32 = jnp.float32


def _mm(x, w):
    return lax.dot_general(x, w, (((1,), (1,)), ((), ())),
                           preferred_element_type=_f32)


def _headmask():
    return (lax.broadcasted_iota(jnp.int32, (HEADS, HID), 1) // DH ==
            lax.broadcasted_iota(jnp.int32, (HEADS, HID), 0)).astype(_f32)


def _embed_body(x_ref, w_ref, b_ref, o_ref):
    o_ref[...] = jax.nn.relu(_mm(x_ref[...], w_ref[...]) + b_ref[...])


def _embed(x_p, w_p, b2):
    return pl.pallas_call(
        _embed_body,
        out_shape=jax.ShapeDtypeStruct((N, HID), _f32),
    )(x_p, w_p, b2)


def _pre_body(h_ref, w_ref, asv_ref, adv_ref, hwx_ref, adp_ref, m_ref):
    h = h_ref[...]
    w = w_ref[...]
    mask = _headmask()
    a_s = asv_ref[...] * mask
    a_d = adv_ref[...] * mask
    hw = _mm(h, w)
    asn = _mm(hw, a_s)
    adn = _mm(hw, a_d)
    hwx_ref[...] = jnp.concatenate(
        [hw, asn, adn, jnp.zeros((N, EW - HID - 8), _f32)], axis=1)
    adw = jnp.concatenate([adn, jnp.zeros((N, 16 - HEADS), _f32)], axis=1)
    adp_ref[...] = jnp.concatenate([adw, jnp.zeros((NP - N, 16), _f32)],
                                   axis=0)
    m4 = jnp.max(asn, axis=0) + jnp.max(adn, axis=0)
    m4 = jnp.where(m4 > 0, m4, m4 * 0.2)
    mpad = jnp.concatenate([m4, jnp.zeros((HID - HEADS,), _f32)])
    m_ref[...] = jnp.broadcast_to(mpad[None, :], (8, HID))


def _pre(h, w, asv, adv):
    return pl.pallas_call(
        _pre_body,
        out_shape=(
            jax.ShapeDtypeStruct((N, EW), _f32),
            jax.ShapeDtypeStruct((NP, 16), _f32),
            jax.ShapeDtypeStruct((8, HID), _f32),
        ),
    )(h, w, asv, adv)


def _post_body(acc_ref, hwx_ref, h_ref, m_ref, g_ref, b_ref, bias_ref, o_ref):
    mask = _headmask()
    num = acc_ref[0, :N, :HID] + acc_ref[1, :N, :HID]
    den = acc_ref[0, :N, HID:HID + 4] + acc_ref[1, :N, HID:HID + 4]
    hw = hwx_ref[:, :HID]
    as4 = hwx_ref[:, HID:HID + 4]
    ad4 = hwx_ref[:, HID + 4:HID + 8]
    m4 = m_ref[0:1, 0:HEADS]
    x = as4 + ad4
    selfw = jnp.exp(jnp.where(x > 0, x, x * 0.2) - m4)
    den = den + selfw
    num = num + hw * lax.dot_general(selfw, mask, (((1,), (0,)), ((), ())),
                                     preferred_element_type=_f32)
    denr = lax.dot_general(den + 1e-16, mask, (((1,), (0,)), ((), ())),
                           preferred_element_type=_f32)
    out = num / denr + bias_ref[...]
    mu = jnp.mean(out, axis=1, keepdims=True)
    var = jnp.mean((out - mu) ** 2, axis=1, keepdims=True)
    ln = (out - mu) / jnp.sqrt(var + 1e-5) * g_ref[...] + b_ref[...]
    o_ref[...] = h_ref[...] + jax.nn.relu(ln)


def _post(acc, hwx, h, m, ln_g, ln_b, bias):
    return pl.pallas_call(
        _post_body,
        out_shape=jax.ShapeDtypeStruct((N, HID), _f32),
    )(acc, hwx, h, m, ln_g, ln_b, bias)


def _head_body(h_ref, batch_ref, o1w_ref, o1b_ref, o2w_ref, o2b_ref,
               seq_ref, seqw_ref, seqb_ref, strw_ref, strb_ref,
               v1w_ref, v1b_ref, o1aw_ref, o1ab_ref,
               v2w_ref, v2b_ref, o2aw_ref, o2ab_ref,
               p1w_ref, p1b_ref, p2w_ref, p2b_ref, p3w_ref, p3b_ref, o_ref):
    batch = batch_ref[...]
    rows = lax.broadcasted_iota(jnp.int32, (B, N), 0)
    oh = (batch == rows).astype(_f32)
    sums = lax.dot_general(oh, h_ref[...], (((1,), (0,)), ((), ())),
                           preferred_element_type=_f32)
    cnt = jnp.sum(oh, axis=1, keepdims=True)
    g = sums / jnp.maximum(cnt, 1.0)
    g = jax.nn.relu(_mm(g, o1w_ref[...]) + o1b_ref[...])
    struct_emb = _mm(g, o2w_ref[...]) + o2b_ref[...]
    seq_h = _mm(seq_ref[...], seqw_ref[...]) + seqb_ref[...]
    struct_h = _mm(struct_emb, strw_ref[...]) + strb_ref[...]
    seq_att = _mm(_mm(struct_h, v1w_ref[...]) + v1b_ref[...],
                  o1aw_ref[...]) + o1ab_ref[...]
    str_att = _mm(_mm(seq_h, v2w_ref[...]) + v2b_ref[...],
                  o2aw_ref[...]) + o2ab_ref[...]
    comb = jnp.concatenate([seq_att, str_att, seq_h, struct_h], axis=1)
    z = jax.nn.relu(_mm(comb, p1w_ref[...]) + p1b_ref[...])
    z = jax.nn.relu(_mm(z, p2w_ref[...]) + p2b_ref[...])
    res = jnp.sum(z * p3w_ref[...], axis=1, keepdims=True) + p3b_ref[...]
    o_ref[...] = jnp.broadcast_to(res, (B, HID))


def _head(*args):
    return pl.pallas_call(
        _head_body,
        out_shape=jax.ShapeDtypeStruct((B, HID), _f32),
    )(*args)


_mesh = plsc.VectorSubcoreMesh(core_axis_name="c", subcore_axis_name="s",
                               num_cores=2, num_subcores=16)


def _edge_body(hwx_hbm, ad_hbm, m_hbm, src_hbm, dst_hbm, acc_out,
               src_t, dst_t, g_t, ad_t, m_t, acc_sh, sem):
    c = lax.axis_index("c")
    s = lax.axis_index("s")
    slab = c * 16 + s
    pltpu.sync_copy(m_hbm, m_t)

    def _zrow(i, _):
        for q in range(EW // 16):
            g_t[i, pl.ds(q * 16, 16)] = jnp.zeros((16,), _f32)
        return 0
    lax.fori_loop(0, C, _zrow, 0)
    base = pl.multiple_of(s * ROWS_PER_TILE, 8)
    for k in range(4):
        pltpu.sync_copy(g_t, acc_sh.at[pl.ds(base + k * C, C)])
    pltpu.sync_copy(g_t.at[pl.ds(0, ROWS_PER_TILE - 4 * C)],
                    acc_sh.at[pl.ds(base + 4 * C, ROWS_PER_TILE - 4 * C)])
    plsc.subcore_barrier()

    def _chunk(j, _):
        pltpu.sync_copy(src_hbm.at[slab, j], src_t.at[0])
        pltpu.sync_copy(dst_hbm.at[slab, j], dst_t.at[0])
        pltpu.async_copy(ad_hbm.at[dst_t.at[0]], ad_t, sem).wait()
        pltpu.async_copy(hwx_hbm.at[src_t.at[0]], g_t, sem).wait()
        mvec = m_t[...]
        for hh in range(HEADS):
            mh = mvec[hh]
            colw = jnp.full((16,), HID + hh, jnp.int32)
            colh = jnp.full((16,), hh, jnp.int32)
            for gg in range(C // 16):
                rows = gg * 16 + lax.iota(jnp.int32, 16)
                a_s = plsc.load_gather(g_t, [rows, colw])
                a_d = plsc.load_gather(ad_t, [rows, colh])
                xx = a_s + a_d
                ee = jnp.where(xx > 0, xx, xx * 0.2)
                wv = jnp.exp(ee - mh)
                plsc.store_scatter(g_t, [rows, colw], wv)

        def _edge(i, _):
            wvec = g_t[i, pl.ds(HID, 16)]
            for hh in range(HEADS):
                wsc = wvec[hh]
                for q in range(2):
                    cc = hh * DH + q * 16
                    g_t[i, pl.ds(cc, 16)] = g_t[i, pl.ds(cc, 16)] * wsc
            return 0
        lax.fori_loop(0, C, _edge, 0)

        pltpu.sync_copy(g_t, acc_sh.at[dst_t.at[0]], add=True)
        return 0
    lax.fori_loop(0, NCHUNK, _chunk, 0)
    plsc.subcore_barrier()
    for k in range(4):
        pltpu.sync_copy(acc_sh.at[pl.ds(base + k * C, C)], g_t)
        pltpu.sync_copy(g_t, acc_out.at[c, pl.ds(base + k * C, C)])
    pltpu.sync_copy(acc_sh.at[pl.ds(base + 4 * C, ROWS_PER_TILE - 4 * C)],
                    g_t.at[pl.ds(0, ROWS_PER_TILE - 4 * C)])
    pltpu.sync_copy(g_t.at[pl.ds(0, ROWS_PER_TILE - 4 * C)],
                    acc_out.at[c, pl.ds(base + 4 * C, ROWS_PER_TILE - 4 * C)])


_edge_call = functools.partial(
    pl.kernel,
    out_type=jax.ShapeDtypeStruct((2, NP, EW), _f32),
    mesh=_mesh,
    compiler_params=pltpu.CompilerParams(use_tc_tiling_on_sc=False,
                                         needs_layout_passes=False),
    scratch_types=[
        pltpu.VMEM((1, C), jnp.int32),
        pltpu.VMEM((1, C), jnp.int32),
        pltpu.VMEM((C, EW), _f32),
        pltpu.VMEM((C, 16), _f32),
        pltpu.VMEM((16,), _f32),
        pltpu.VMEM_SHARED((NP, EW), _f32),
        pltpu.SemaphoreType.DMA,
    ],
)(_edge_body)


def kernel(seq_embedding, x, edge_index, batch, embed_W, embed_b, gat_W,
           gat_as, gat_ad, gat_b, ln_g, ln_b, out1_W, out1_b, out2_W, out2_b,
           seq_W, seq_b, str_W, str_b, mha1_Wi, mha1_bi, mha1_Wo, mha1_bo,
           mha2_Wi, mha2_bi, mha2_Wo, mha2_bo, p1_W, p1_b, p2_W, p2_b,
           p3_W, p3_b):
    x_p = jnp.pad(x, ((0, 0), (0, 7)))
    w_p = jnp.pad(embed_W, ((0, 0), (0, 7)))
    eir = edge_index.reshape(2, NTILES, NCHUNK, CREAL)
    srcp = jnp.pad(eir[0], ((0, 0), (0, 0), (0, C - CREAL)),
                   constant_values=0)
    dstp = jnp.pad(eir[1], ((0, 0), (0, 0), (0, C - CREAL)),
                   constant_values=DUMMY_DST)
    asv = gat_as.reshape(LAYERS, 1, HID)
    adv = gat_ad.reshape(LAYERS, 1, HID)

    h = _embed(x_p, w_p, embed_b.reshape(1, HID))
    for i in range(LAYERS):
        hwx, adp, m = _pre(h, gat_W[i], asv[i], adv[i])
        acc = _edge_call(hwx, adp, m[0, :16], srcp, dstp)
        h = _post(acc, hwx, h, m, ln_g[i].reshape(1, HID),
                  ln_b[i].reshape(1, HID), gat_b[i].reshape(1, HID))

    out = _head(h, batch.reshape(1, N),
                out1_W, out1_b.reshape(1, HID), out2_W, out2_b.reshape(1, HID),
                seq_embedding, seq_W, seq_b.reshape(1, HD),
                str_W, str_b.reshape(1, HD),
                mha1_Wi[2 * HD:], mha1_bi[2 * HD:].reshape(1, HD),
                mha1_Wo, mha1_bo.reshape(1, HD),
                mha2_Wi[2 * HD:], mha2_bi[2 * HD:].reshape(1, HD),
                mha2_Wo, mha2_bo.reshape(1, HD),
                p1_W, p1_b.reshape(1, HD), p2_W, p2_b.reshape(1, HD // 2),
                p3_W.reshape(1, HD // 2), p3_b.reshape(1, 1))
    return out[:, 0]
